# baseline (speedup 1.0000x reference)
import functools

import jax
import jax.numpy as jnp
from jax import lax
from jax.experimental import pallas as pl
from jax.experimental.pallas import tpu as pltpu
from jax.experimental.pallas import tpu_sc as plsc

B = 4096
D = 128
NEB = 2048
NBLK = 50
NTAB = NBLK * NEB
BB = 1024
NBB = B // BB
K = 10
NSUB = NEB // 128


def _proj_kernel(sq_ref, rq_ref, oq_ref, wqs_ref, wqr_ref, wqo_ref,
                 wke_ref, wve_ref, wkr_ref, wvr_ref, ls_ref,
                 p_ref, u_ref):
    scale = jnp.exp(ls_ref[0, 0])
    qs = [sq_ref[...], rq_ref[...], oq_ref[...]]
    wq = [wqs_ref[...], wqr_ref[...], wqo_ref[...]]
    wk = [wke_ref[...], wkr_ref[...], wke_ref[...]]
    wv = [wve_ref[...], wvr_ref[...], wve_ref[...]]
    dn = (((0,), (0,)), ((), ()))
    for s in range(3):
        q = qs[s]
        qn = q * lax.rsqrt(jnp.maximum(jnp.sum(q * q, -1, keepdims=True),
                                       1e-30))
        mk = lax.dot_general(wq[s], wk[s], dn,
                             preferred_element_type=jnp.float32)
        mv = lax.dot_general(wq[s], wv[s], dn,
                             preferred_element_type=jnp.float32)
        p_ref[s * B:(s + 1) * B, :] = scale * jnp.dot(
            qn, mk, preferred_element_type=jnp.float32)
        u_ref[s * B:(s + 1) * B, :] = jnp.dot(
            qn, mv, preferred_element_type=jnp.float32)


def _extract10(gm, gbase, bb):
    l128 = lax.broadcasted_iota(jnp.int32, (bb, 128), 1)
    l16 = lax.broadcasted_iota(jnp.int32, (bb, 16), 1)
    blkv = jnp.full((bb, 16), -3e38, jnp.float32)
    blkg = jnp.zeros((bb, 16), jnp.int32)
    for t in range(K):
        m = jnp.max(gm, axis=1)
        a = jnp.argmax(gm, axis=1).astype(jnp.int32)
        gm = jnp.where(l128 == a[:, None], -3e38, gm)
        blkv = jnp.where(l16 == t, m[:, None], blkv)
        blkg = jnp.where(l16 == t, (gbase + a)[:, None], blkg)
    return blkv, blkg


def _merge10(av, ag, bv, bg, bb):
    cv = jnp.concatenate([av, bv], axis=1)
    cg = jnp.concatenate([ag, bg], axis=1)
    l32 = lax.broadcasted_iota(jnp.int32, (bb, 32), 1)
    l16 = lax.broadcasted_iota(jnp.int32, (bb, 16), 1)
    nv = jnp.full((bb, 16), -3e38, jnp.float32)
    ng = jnp.zeros((bb, 16), jnp.int32)
    for t in range(K):
        m = jnp.max(cv, axis=1)
        a = jnp.argmax(cv, axis=1).astype(jnp.int32)
        g = jnp.sum(jnp.where(l32 == a[:, None], cg, 0), axis=1)
        cv = jnp.where(l32 == a[:, None], -3e38, cv)
        nv = jnp.where(l16 == t, m[:, None], nv)
        ng = jnp.where(l16 == t, g[:, None], ng)
    return nv, ng


def _screen_kernel(pall_ref, ktab_ref, kgn_ref, gs_ref, gr_ref, go_ref,
                   runv_ref, rung_ref):
    j = pl.program_id(0)
    b = pl.program_id(1)
    t = ktab_ref[...]
    kn = t * lax.rsqrt(jnp.maximum(jnp.sum(t * t, -1, keepdims=True), 1e-30))
    kgn_ref[...] = kn

    lim = jnp.where(j == NBLK - 2, 100000 - (NBLK - 2) * NEB,
                    jnp.where(j == NBLK - 1, 1000, NEB))
    colio = lax.broadcasted_iota(jnp.int32, (BB, NEB), 1)
    colmask = colio < lim

    def tile_topk(p):
        lg = lax.dot_general(p, kn, (((1,), (1,)), ((), ())),
                             preferred_element_type=jnp.float32)
        lg = jnp.where(colmask, lg, -3e38)
        gm = lg[:, :128]
        for k in range(1, NSUB):
            gm = jnp.maximum(gm, lg[:, k * 128:(k + 1) * 128])
        return _extract10(gm, j * NEB, BB)

    @pl.when(j < NBLK - 1)
    def _():
        for s, (prow, runrow) in enumerate(((0, 0), (2 * B, B))):
            bv, bg = tile_topk(pall_ref[pl.ds(prow + b * BB, BB), :])
            rows = pl.ds(runrow + b * BB, BB)
            pv = jnp.where(j == 0, -3e38, runv_ref[rows, :])
            pg = jnp.where(j == 0, 0, rung_ref[rows, :])
            nv, ng = _merge10(pv, pg, bv, bg, BB)
            runv_ref[rows, :] = nv
            rung_ref[rows, :] = ng

    @pl.when(j == NBLK - 1)
    def _():
        bv, bg = tile_topk(pall_ref[pl.ds(B + b * BB, BB), :])
        gr_ref[:, :16] = bg

    gs_ref[:, :16] = rung_ref[pl.ds(b * BB, BB), :]
    go_ref[:, :16] = rung_ref[pl.ds(B + b * BB, BB), :]


def _stage_pa(subj_q, rel_q, obj_q, entity_embeddings, relation_embeddings,
              Wq_subj, Wq_rel, Wq_obj, Wk_e, Wv_e, Wk_r, Wv_r, logit_scale):
    ktab = jnp.concatenate([
        jnp.pad(entity_embeddings, ((0, (NBLK - 1) * NEB - 100000), (0, 0))),
        jnp.pad(relation_embeddings, ((0, NEB - 1000), (0, 0))),
    ], axis=0)

    p_all, u_all = pl.pallas_call(
        _proj_kernel,
        in_specs=[
            pl.BlockSpec((B, D), lambda: (0, 0)),
            pl.BlockSpec((B, D), lambda: (0, 0)),
            pl.BlockSpec((B, D), lambda: (0, 0)),
        ] + [pl.BlockSpec((D, D), lambda: (0, 0))] * 7 + [
            pl.BlockSpec(memory_space=pltpu.SMEM),
        ],
        out_specs=(pl.BlockSpec((3 * B, D), lambda: (0, 0)),
                   pl.BlockSpec((3 * B, D), lambda: (0, 0))),
        out_shape=(jax.ShapeDtypeStruct((3 * B, D), jnp.float32),
                   jax.ShapeDtypeStruct((3 * B, D), jnp.float32)),
    )(subj_q, rel_q, obj_q, Wq_subj, Wq_rel, Wq_obj,
      Wk_e, Wv_e, Wk_r, Wv_r,
      jnp.reshape(logit_scale, (1, 1)))

    kgn, gs, gr, go = pl.pallas_call(
        _screen_kernel,
        grid=(NBLK, NBB),
        in_specs=[
            pl.BlockSpec((3 * B, D), lambda j, b: (0, 0)),
            pl.BlockSpec((NEB, D), lambda j, b: (j, 0)),
        ],
        out_specs=(
            pl.BlockSpec((NEB, D), lambda j, b: (j, 0)),
            pl.BlockSpec((BB, 128), lambda j, b: (b, 0)),
            pl.BlockSpec((BB, 128), lambda j, b: (b, 0)),
            pl.BlockSpec((BB, 128), lambda j, b: (b, 0)),
        ),
        out_shape=(
            jax.ShapeDtypeStruct((NTAB, D), jnp.float32),
            jax.ShapeDtypeStruct((B, 128), jnp.int32),
            jax.ShapeDtypeStruct((B, 128), jnp.int32),
            jax.ShapeDtypeStruct((B, 128), jnp.int32),
        ),
        scratch_shapes=[
            pltpu.VMEM((2 * B, 16), jnp.float32),
            pltpu.VMEM((2 * B, 16), jnp.int32),
        ],
    )(p_all, ktab)
    return p_all, u_all, kgn, gs, gr, go


NW = 32
QPW = 3 * B // NW
CH = 48
NCH = QPW // CH
NC = 160
ENT_LIMIT = 100000
REL_LIMIT = (NBLK - 1) * NEB + 1000


def _expand_kernel(gs_ref, gr_ref, go_ref, idx_ref):
    lane16 = lax.broadcasted_iota(jnp.int32, (B, 16), 1)
    for s, ref in enumerate((gs_ref, gr_ref, go_ref)):
        g = ref[:, :16]
        for i in range(K):
            gi = jnp.sum(jnp.where(lane16 == i, g, 0), axis=1)
            for k in range(16):
                idx_ref[s * B:(s + 1) * B, i * 16 + k] = gi + 128 * k


def _sc_body(ktab, pall, uall, idxall, out,
             p_v, u_v, ix_v, idx_v, widx_v, rows_v, wrows_v, lg_v,
             ff_v, fi_v, sc_v, sem):
    wid = lax.axis_index("s") * 2 + lax.axis_index("c")
    qbase = wid * QPW
    lane = lax.iota(jnp.int32, 16)
    zero16f = jnp.zeros((16,), jnp.float32)

    def bcast_sum(v):
        acc = v
        for o in (8, 4, 2, 1):
            ff_v[pl.ds(0, 16)] = acc
            ff_v[pl.ds(16, 16)] = acc
            acc = acc + ff_v[pl.ds(o, 16)]
        return acc

    def bcast_maxid(v, ids):
        for o in (8, 4, 2, 1):
            ff_v[pl.ds(0, 16)] = v
            ff_v[pl.ds(16, 16)] = v
            fi_v[pl.ds(0, 16)] = ids
            fi_v[pl.ds(16, 16)] = ids
            vr = ff_v[pl.ds(o, 16)]
            ir = fi_v[pl.ds(o, 16)]
            take = (vr > v) | ((vr == v) & (ir < ids))
            v = jnp.where(take, vr, v)
            ids = jnp.where(take, ir, ids)
        return v, ids

    def chunk_body(ch, _):
        cbase = qbase + ch * CH
        pltpu.sync_copy(pall.at[pl.ds(cbase, CH)], p_v)
        pltpu.sync_copy(uall.at[pl.ds(cbase, CH)], u_v)
        pltpu.sync_copy(idxall.at[pl.ds(cbase, CH)], ix_v)

        def query_body(q, _):
            qq = cbase + q
            is_rel = jnp.logical_and(qq >= B, qq < 2 * B)
            limit = jnp.where(is_rel, REL_LIMIT, ENT_LIMIT)

            def cp_body(i, _):
                idx_v[pl.ds(i * 16, 16)] = ix_v[q, pl.ds(i * 16, 16)]
                return 0
            lax.fori_loop(0, K, cp_body, 0)
            pltpu.async_copy(ktab.at[idx_v], rows_v, sem).wait()

            pc = [p_v[q, pl.ds(c * 16, 16)] for c in range(8)]

            def grp_body(i, dots):
                def row_body(r, dots):
                    row = i * 16 + r
                    acc = rows_v[row, pl.ds(0, 16)] * pc[0]
                    for c in range(1, 8):
                        acc = acc + rows_v[row, pl.ds(c * 16, 16)] * pc[c]
                    tot = bcast_sum(acc)
                    return jnp.where(lane == r, tot, dots)
                dots = lax.fori_loop(0, 16, row_body, dots)
                ids = idx_v[pl.ds(i * 16, 16)]
                dots = jnp.where(ids < limit, dots, -3e38)
                lg_v[pl.ds(i * 16, 16)] = dots
                return dots * 0.0
            lax.fori_loop(0, K, grp_body, zero16f)

            def top_body(t, carry):
                wv, wi = carry
                mv = lg_v[pl.ds(0, 16)]
                mi = idx_v[pl.ds(0, 16)]
                def pre_body(i, c):
                    mv, mi = c
                    v2 = lg_v[pl.ds(i * 16, 16)]
                    i2 = idx_v[pl.ds(i * 16, 16)]
                    take = (v2 > mv) | ((v2 == mv) & (i2 < mi))
                    return (jnp.where(take, v2, mv), jnp.where(take, i2, mi))
                mv, mi = lax.fori_loop(1, K, pre_body, (mv, mi))
                mvb, mib = bcast_maxid(mv, mi)
                def mask_body(i, _):
                    ids = idx_v[pl.ds(i * 16, 16)]
                    cur = lg_v[pl.ds(i * 16, 16)]
                    lg_v[pl.ds(i * 16, 16)] = jnp.where(ids == mib, -3e38, cur)
                    return 0
                lax.fori_loop(0, K, mask_body, 0)
                wv = jnp.where(lane == t, mvb, wv)
                wi = jnp.where(lane == t, mib, wi)
                return (wv, wi)
            wv, wi = lax.fori_loop(
                0, K, top_body,
                (jnp.full((16,), -3e38, jnp.float32), jnp.zeros((16,), jnp.int32)))

            vmask = jnp.where(lane < K, wv, -3e38)
            mxb, _ = bcast_maxid(vmask, jnp.zeros((16,), jnp.int32))
            e = jnp.exp(vmask - mxb)
            e = jnp.where(lane < K, e, 0.0)
            attn = e / bcast_sum(e)

            widx_v[...] = wi
            pltpu.async_copy(ktab.at[widx_v], wrows_v, sem).wait()
            uc = [u_v[q, pl.ds(c * 16, 16)] for c in range(8)]

            def vd_body(w, vd):
                acc = wrows_v[w, pl.ds(0, 16)] * uc[0]
                for c in range(1, 8):
                    acc = acc + wrows_v[w, pl.ds(c * 16, 16)] * uc[c]
                tot = bcast_sum(acc)
                return jnp.where(lane == w, tot, vd)
            vd = lax.fori_loop(0, K, vd_body, zero16f)

            score = bcast_sum(attn * vd)
            base = (q // 16) * 16
            cur = sc_v[pl.ds(base, 16)]
            sc_v[pl.ds(base, 16)] = jnp.where(lane == q % 16, score, cur)
            return 0
        lax.fori_loop(0, CH, query_body, 0)
        pltpu.sync_copy(sc_v, out.at[pl.ds(cbase, CH)])
        return 0
    lax.fori_loop(0, NCH, chunk_body, 0)


def _stage_c(ktab_n, p_all, u_all, gs, gr, go):
    idx_all = pl.pallas_call(
        _expand_kernel,
        in_specs=[pl.BlockSpec((B, 128), lambda: (0, 0))] * 3,
        out_specs=pl.BlockSpec((3 * B, NC), lambda: (0, 0)),
        out_shape=jax.ShapeDtypeStruct((3 * B, NC), jnp.int32),
    )(gs, gr, go)
    mesh = plsc.VectorSubcoreMesh(core_axis_name="c", subcore_axis_name="s")
    fn = functools.partial(
        pl.kernel, mesh=mesh,
        out_type=jax.ShapeDtypeStruct((3 * B,), jnp.float32),
        scratch_types=[
            pltpu.VMEM((CH, 128), jnp.float32),
            pltpu.VMEM((CH, 128), jnp.float32),
            pltpu.VMEM((CH, NC), jnp.int32),
            pltpu.VMEM((NC,), jnp.int32),
            pltpu.VMEM((16,), jnp.int32),
            pltpu.VMEM((NC, 128), jnp.float32),
            pltpu.VMEM((16, 128), jnp.float32),
            pltpu.VMEM((NC,), jnp.float32),
            pltpu.VMEM((32,), jnp.float32),
            pltpu.VMEM((32,), jnp.int32),
            pltpu.VMEM((CH,), jnp.float32),
            pltpu.SemaphoreType.DMA,
        ],
    )(_sc_body)
    return fn(ktab_n, p_all, u_all, idx_all)


def kernel(subj_q, rel_q, obj_q, entity_embeddings, relation_embeddings,
           Wq_subj, Wq_rel, Wq_obj, Wk_e, Wv_e, Wk_r, Wv_r, logit_scale):
    p_all, u_all, kgn, gs, gr, go = _stage_pa(
        subj_q, rel_q, obj_q, entity_embeddings, relation_embeddings,
        Wq_subj, Wq_rel, Wq_obj, Wk_e, Wv_e, Wk_r, Wv_r, logit_scale)
    score = _stage_c(kgn, p_all, u_all, gs, gr, go)
    return score.reshape(3, B)
